# baseline (speedup 1.0000x reference)
import functools

import numpy as np

import jax
import jax.numpy as jnp
from jax import lax
from jax.experimental import pallas as pl
from jax.experimental.pallas import tpu as pltpu
from jax.experimental.pallas import tpu_sc as plsc

_NC = 2
_NS = 16
_NW = _NC * _NS
_CH = 128
_SB = 8
_HI = -65536


def _node_precompute(node, Wn, bn2, We_top_p):
    n, d = node.shape
    blk = 1000

    def body(node_ref, wn_ref, bn_ref, wet_ref, out_ref):
        xb = node_ref[...].astype(jnp.bfloat16)
        wn = wn_ref[...].astype(jnp.bfloat16)
        h = jnp.dot(xb, wn, preferred_element_type=jnp.float32) + bn_ref[...]
        h = jnp.where(h >= 0, h, 0.01 * h)
        wet = wet_ref[...].astype(jnp.bfloat16)
        p = jnp.dot(h.astype(jnp.bfloat16), wet,
                    preferred_element_type=jnp.float32)
        a = p[:, : d // 2].astype(jnp.bfloat16).astype(jnp.float32)
        b = p[:, d // 2:].astype(jnp.bfloat16).astype(jnp.float32)
        ai = jax.lax.bitcast_convert_type(a, jnp.int32)
        bi = jax.lax.bitcast_convert_type(b, jnp.int32)
        out_ref[...] = jax.lax.shift_right_logical(ai, 16) | (bi & jnp.int32(_HI))

    return pl.pallas_call(
        body,
        grid=(n // blk,),
        in_specs=[
            pl.BlockSpec((blk, d), lambda i: (i, 0)),
            pl.BlockSpec((d, d), lambda i: (0, 0)),
            pl.BlockSpec((1, d), lambda i: (0, 0)),
            pl.BlockSpec((d, d), lambda i: (0, 0)),
        ],
        out_specs=pl.BlockSpec((blk, d // 2), lambda i: (i, 0)),
        out_shape=jax.ShapeDtypeStruct((n, d // 2), jnp.int32),
    )(node, Wn, bn2, We_top_p)


def _gather_rows(table, idx2d):
    n, dp = table.shape
    n_chunks, ch = idx2d.shape
    assert ch == _CH and n_chunks % (_SB * _NW) == 0
    n_sb = n_chunks // _SB
    per_w = n_sb // _NW
    mesh = plsc.VectorSubcoreMesh(core_axis_name="c", subcore_axis_name="s")

    @functools.partial(
        pl.kernel,
        out_type=jax.ShapeDtypeStruct((n_chunks * _CH, dp), jnp.int32),
        mesh=mesh,
        scratch_types=[
            pltpu.VMEM((_SB, _CH), jnp.int32),
            pltpu.VMEM((_SB, _CH), jnp.int32),
            pltpu.VMEM((_CH, dp), jnp.int32),
            pltpu.VMEM((_CH, dp), jnp.int32),
            pltpu.SemaphoreType.DMA,
            pltpu.SemaphoreType.DMA,
            pltpu.SemaphoreType.DMA,
            pltpu.SemaphoreType.DMA,
            pltpu.SemaphoreType.DMA,
            pltpu.SemaphoreType.DMA,
        ],
    )
    def k(table_hbm, idx_hbm, out_hbm, idx0, idx1, rows0, rows1,
          si0, si1, sg0, sg1, sw0, sw1):
        wid = lax.axis_index("s") * _NC + lax.axis_index("c")
        idx_b = (idx0, idx1)
        rows_b = (rows0, rows1)
        si = (si0, si1)
        sg = (sg0, sg1)
        sw = (sw0, sw1)

        def sbase(j):
            return (j * _NW + wid) * _SB

        pltpu.async_copy(idx_hbm.at[pl.ds(sbase(0), _SB)], idx_b[0], si[0])
        for j in range(per_w):
            sp = j % 2
            pltpu.make_async_copy(
                idx_hbm.at[pl.ds(0, _SB)], idx_b[sp], si[sp]).wait()
            for kk in range(_SB):
                i = j * _SB + kk
                b = kk % 2
                c = sbase(j) + kk
                if i >= 2:
                    pltpu.make_async_copy(
                        rows_b[b], out_hbm.at[pl.ds(0, _CH)], sw[b]).wait()
                pltpu.async_copy(table_hbm.at[idx_b[sp].at[kk]],
                                 rows_b[b], sg[b])
                if kk == 2 and j + 1 < per_w:
                    pltpu.async_copy(
                        idx_hbm.at[pl.ds(sbase(j + 1), _SB)],
                        idx_b[1 - sp], si[1 - sp])
                if i >= 1:
                    c_prev = c - 1 if kk > 0 else (sbase(j - 1) + _SB - 1)
                    pltpu.make_async_copy(
                        table_hbm.at[idx_b[sp].at[kk]], rows_b[1 - b],
                        sg[1 - b]).wait()
                    pltpu.async_copy(
                        rows_b[1 - b], out_hbm.at[pl.ds(c_prev * _CH, _CH)],
                        sw[1 - b])
        last_c = sbase(per_w - 1) + _SB - 1
        pltpu.make_async_copy(
            table_hbm.at[idx_b[(per_w - 1) % 2].at[_SB - 1]],
            rows_b[1], sg[1]).wait()
        pltpu.async_copy(rows_b[1], out_hbm.at[pl.ds(last_c * _CH, _CH)],
                         sw[1])
        for b in range(2):
            pltpu.make_async_copy(
                rows_b[b], out_hbm.at[pl.ds(0, _CH)], sw[b]).wait()

    return k(table, idx2d)


def _edge_compute(G, edge, We_bot_p, be2_p, Wt_p, bt2, e_pad):
    e, d = edge.shape
    blk = 1280

    def body(g_ref, e_ref, web_ref, be_ref, wt_ref, bt_ref, out_ref):
        eb = e_ref[...].astype(jnp.bfloat16)
        web = web_ref[...].astype(jnp.bfloat16)
        m = jnp.dot(eb, web, preferred_element_type=jnp.float32)
        g = g_ref[...]
        ge = jax.lax.bitcast_convert_type(g << 16, jnp.float32)
        go = jax.lax.bitcast_convert_type(g & jnp.int32(_HI), jnp.float32)
        m = m + jnp.concatenate([ge, go], axis=1) + be_ref[...]
        m = jnp.where(m >= 0, m, 0.01 * m)
        wt = wt_ref[...].astype(jnp.bfloat16)
        out_ref[...] = jnp.dot(m.astype(jnp.bfloat16), wt,
                               preferred_element_type=jnp.float32) + bt_ref[...]

    return pl.pallas_call(
        body,
        grid=(e // blk,),
        in_specs=[
            pl.BlockSpec((blk, d // 2), lambda i: (i, 0)),
            pl.BlockSpec((blk, d), lambda i: (i, 0)),
            pl.BlockSpec((d, d), lambda i: (0, 0)),
            pl.BlockSpec((1, d), lambda i: (0, 0)),
            pl.BlockSpec((d, d), lambda i: (0, 0)),
            pl.BlockSpec((1, d), lambda i: (0, 0)),
        ],
        out_specs=pl.BlockSpec((blk, d), lambda i: (i, 0)),
        out_shape=jax.ShapeDtypeStruct((e_pad, d), jnp.float32),
    )(G, edge, We_bot_p, be2_p, Wt_p, bt2)


def _scatter_add_relu(ac, dst2d, n):
    e_pad, d = ac.shape
    dh = d // _NC
    n_chunks, ch = dst2d.shape
    assert ch == _CH and n_chunks * _CH == e_pad
    assert n_chunks % (_SB * _NS) == 0
    n_sb = n_chunks // _SB
    per_t = n_sb // _NS
    wb = 80
    n_blk = n // wb
    assert n_blk * wb == n
    per_wb = (n_blk + _NS - 1) // _NS
    mesh = plsc.VectorSubcoreMesh(core_axis_name="c", subcore_axis_name="s")

    @functools.partial(
        pl.kernel,
        out_type=jax.ShapeDtypeStruct((n, d), jnp.float32),
        mesh=mesh,
        scratch_types=[
            pltpu.VMEM((_SB, _CH), jnp.int32),
            pltpu.VMEM((_SB, _CH), jnp.int32),
            pltpu.VMEM((_CH, dh), jnp.float32),
            pltpu.VMEM((_CH, dh), jnp.float32),
            pltpu.VMEM((wb, dh), jnp.float32),
            pltpu.VMEM_SHARED((n + _SB, dh), jnp.float32),
            pltpu.SemaphoreType.DMA,
            pltpu.SemaphoreType.DMA,
            pltpu.SemaphoreType.DMA,
            pltpu.SemaphoreType.DMA,
            pltpu.SemaphoreType.DMA,
            pltpu.SemaphoreType.DMA,
        ],
    )
    def k(ac_hbm, dst_hbm, out_hbm, idx0, idx1, rows0, rows1, buf_v, acc_sh,
          si0, si1, sl0, sl1, sa0, sa1):
        cid = lax.axis_index("c")
        sid = lax.axis_index("s")
        idx_b = (idx0, idx1)
        rows_b = (rows0, rows1)
        si = (si0, si1)
        sl = (sl0, sl1)
        sa = (sa0, sa1)

        def zbody(i, carry):
            for j in range(dh // 16):
                buf_v[i, pl.ds(j * 16, 16)] = jnp.zeros((16,), jnp.float32)
            return carry

        lax.fori_loop(0, wb, zbody, 0)

        def zcopy(i, carry):
            g = i * _NS + sid

            @pl.when(g < n_blk)
            def _():
                pltpu.sync_copy(buf_v, acc_sh.at[pl.ds(g * wb, wb)])

            return carry

        lax.fori_loop(0, per_wb, zcopy, 0)
        plsc.subcore_barrier()

        def sbase(j):
            return (j * _NS + sid) * _SB

        pltpu.async_copy(dst_hbm.at[pl.ds(sbase(0), _SB)], idx_b[0], si[0])
        for j in range(per_t):
            sp = j % 2
            pltpu.make_async_copy(
                dst_hbm.at[pl.ds(0, _SB)], idx_b[sp], si[sp]).wait()
            for kk in range(_SB):
                i = j * _SB + kk
                b = kk % 2
                c = sbase(j) + kk
                if i >= 2:
                    pltpu.make_async_copy(
                        rows_b[b], acc_sh.at[idx_b[sp].at[kk]], sa[b]).wait()
                pltpu.async_copy(
                    ac_hbm.at[pl.ds(c * _CH, _CH), pl.ds(cid * dh, dh)],
                    rows_b[b], sl[b])
                if kk == 2 and j + 1 < per_t:
                    pltpu.async_copy(
                        dst_hbm.at[pl.ds(sbase(j + 1), _SB)],
                        idx_b[1 - sp], si[1 - sp])
                if i >= 1:
                    if kk > 0:
                        pidx = idx_b[sp].at[kk - 1]
                    else:
                        pidx = idx_b[1 - sp].at[_SB - 1]
                    pltpu.make_async_copy(
                        ac_hbm.at[pl.ds(0, _CH), pl.ds(cid * dh, dh)],
                        rows_b[1 - b], sl[1 - b]).wait()
                    pltpu.async_copy(rows_b[1 - b], acc_sh.at[pidx],
                                     sa[1 - b], add=True)
        lidx = idx_b[(per_t - 1) % 2].at[_SB - 1]
        pltpu.make_async_copy(
            ac_hbm.at[pl.ds(0, _CH), pl.ds(cid * dh, dh)],
            rows_b[1], sl[1]).wait()
        pltpu.async_copy(rows_b[1], acc_sh.at[lidx], sa[1], add=True)
        for b in range(2):
            pltpu.make_async_copy(
                rows_b[b], acc_sh.at[idx_b[0].at[0]], sa[b]).wait()

        plsc.subcore_barrier()

        def wcopy(i, carry):
            g = i * _NS + sid

            @pl.when(g < n_blk)
            def _():
                r0 = g * wb
                pltpu.sync_copy(acc_sh.at[pl.ds(r0, wb)], buf_v)

                def rbody(ii, cc):
                    for j in range(dh // 16):
                        s = pl.ds(j * 16, 16)
                        buf_v[ii, s] = jnp.maximum(buf_v[ii, s], 0.0)
                    return cc

                lax.fori_loop(0, wb, rbody, 0)
                pltpu.sync_copy(buf_v,
                                out_hbm.at[pl.ds(r0, wb), pl.ds(cid * dh, dh)])

            return carry

        lax.fori_loop(0, per_wb, wcopy, 0)

    return k(ac, dst2d)


def kernel(node, edge, edge_index, Wn, bn, We, be, Wa, ba, Wt, bt):
    n, d = node.shape
    e = edge.shape[0]
    n_chunks = e // _CH
    pad_to = _SB * _NW
    n_chunks_pad = ((n_chunks + pad_to - 1) // pad_to) * pad_to
    pad_rows = n_chunks_pad - n_chunks

    src2d = edge_index[0].reshape(n_chunks, _CH)
    dst2d = edge_index[1].reshape(n_chunks, _CH)
    src2d = jnp.concatenate(
        [src2d, jnp.zeros((pad_rows, _CH), jnp.int32)])
    dst2d = jnp.concatenate(
        [dst2d, jnp.full((pad_rows, _CH), n, jnp.int32)])

    perm = np.concatenate([np.arange(0, d, 2), np.arange(1, d, 2)])
    We_top_p = We[:d][:, perm]
    We_bot_p = We[d:][:, perm]
    be_p = be[perm]
    Wt_p = Wt[perm, :]

    P = _node_precompute(node, Wn, bn.reshape(1, d), We_top_p)
    G = _gather_rows(P, src2d)
    ac = _edge_compute(G, edge, We_bot_p, be_p.reshape(1, d), Wt_p,
                       bt.reshape(1, d), n_chunks_pad * _CH)
    return _scatter_add_relu(ac, dst2d, n)
